# baseline (speedup 1.0000x reference)
import jax
import jax.numpy as jnp
from jax import lax
from jax.experimental import pallas as pl
from jax.experimental.pallas import tpu as pltpu
from jax.experimental.pallas import tpu_sc as plsc

VOCAB = 32000
HIDDEN = 768
MAX_POS = 64
LANES = 16
SLICES = HIDDEN // LANES

NUM_CORES = 2
NUM_SUBCORES = 16
NUM_WORKERS = NUM_CORES * NUM_SUBCORES

CHUNK = 32
NBUF = 3


def _sc_embed(ids_hbm, table_hbm, pos_hbm, out_hbm,
              pos_v, idx0, idx1, idx2, rows0, rows1, rows2,
              sem_g0, sem_g1, sem_g2, sem_s0, sem_s1, sem_s2):
    n_tokens = ids_hbm.shape[0]
    per_worker = n_tokens // NUM_WORKERS
    n_chunks = per_worker // CHUNK

    wid = lax.axis_index("s") * NUM_CORES + lax.axis_index("c")
    base = wid * per_worker

    idx = (idx0, idx1, idx2)
    rows = (rows0, rows1, rows2)
    sem_g = (sem_g0, sem_g1, sem_g2)
    sem_s = (sem_s0, sem_s1, sem_s2)

    pltpu.sync_copy(pos_hbm, pos_v)

    for b in range(2):
        pltpu.sync_copy(ids_hbm.at[pl.ds(base + b * CHUNK, CHUNK)], idx[b])
        pltpu.async_copy(table_hbm.at[idx[b]], rows[b], sem_g[b])

    def add_pos(b, c):
        pos_half = (c % (MAX_POS // CHUNK)) * CHUNK

        @plsc.parallel_loop(0, CHUNK, unroll=2)
        def _(t):
            p = pos_half + t
            for j in range(SLICES):
                sl = pl.ds(j * LANES, LANES)
                rows[b][t, sl] += pos_v[p, sl]

    def step(c, b, b2):
        off = base + c * CHUNK

        pltpu.make_async_copy(table_hbm.at[idx[b]], rows[b], sem_g[b]).wait()
        add_pos(b, c)
        pltpu.make_async_copy(
            rows[b], out_hbm.at[pl.ds(off, CHUNK)], sem_s[b]).start()

        @pl.when(c + 2 < n_chunks)
        def _():
            pltpu.sync_copy(
                ids_hbm.at[pl.ds(off + 2 * CHUNK, CHUNK)], idx[b2])

            @pl.when(c >= 1)
            def _():
                pltpu.make_async_copy(
                    rows[b2], out_hbm.at[pl.ds(0, CHUNK)], sem_s[b2]).wait()

            pltpu.async_copy(table_hbm.at[idx[b2]], rows[b2], sem_g[b2])

    def superstep(g, carry):
        c0 = 3 * g
        step(c0, 0, 2)
        step(c0 + 1, 1, 0)
        step(c0 + 2, 2, 1)
        return carry

    lax.fori_loop(0, (n_chunks - 1) // 3, superstep, 0, unroll=False)

    c_last = n_chunks - 1
    pltpu.make_async_copy(table_hbm.at[idx[0]], rows[0], sem_g[0]).wait()
    add_pos(0, c_last)
    pltpu.make_async_copy(
        rows[0], out_hbm.at[pl.ds(base + c_last * CHUNK, CHUNK)],
        sem_s[0]).start()
    for b in range(NBUF):
        pltpu.make_async_copy(
            rows[b], out_hbm.at[pl.ds(0, CHUNK)], sem_s[b]).wait()


def kernel(input_ids, token_embedding, position_embedding):
    batch, seq = input_ids.shape
    n_tokens = batch * seq
    ids_flat = input_ids.reshape(n_tokens).astype(jnp.int32)

    mesh = plsc.VectorSubcoreMesh(core_axis_name="c", subcore_axis_name="s")
    run = pl.kernel(
        _sc_embed,
        mesh=mesh,
        out_type=jax.ShapeDtypeStruct((n_tokens, HIDDEN), jnp.float32),
        scratch_types=[
            pltpu.VMEM((MAX_POS, HIDDEN), jnp.float32),
            pltpu.VMEM((CHUNK,), jnp.int32),
            pltpu.VMEM((CHUNK,), jnp.int32),
            pltpu.VMEM((CHUNK,), jnp.int32),
            pltpu.VMEM((CHUNK, HIDDEN), jnp.float32),
            pltpu.VMEM((CHUNK, HIDDEN), jnp.float32),
            pltpu.VMEM((CHUNK, HIDDEN), jnp.float32),
            pltpu.SemaphoreType.DMA,
            pltpu.SemaphoreType.DMA,
            pltpu.SemaphoreType.DMA,
            pltpu.SemaphoreType.DMA,
            pltpu.SemaphoreType.DMA,
            pltpu.SemaphoreType.DMA,
        ],
    )
    out = run(ids_flat, token_embedding, position_embedding)
    return out.reshape(batch, seq, HIDDEN)
